# baseline (speedup 1.0000x reference)
import functools

import jax
import jax.numpy as jnp
from jax import lax
from jax.experimental import pallas as pl
from jax.experimental.pallas import tpu as pltpu
from jax.experimental.pallas import tpu_sc as plsc

N = 10000
E = 320000
F = 128
C = 64

NC = 2
NS = 16
K = 128
CH = 80
EP = NC * NS * CH * K
NP = 10240
RPT = NP // NS
NB = 2

_HIGH = lax.Precision.HIGHEST


@functools.lru_cache(maxsize=None)
def _sc_scatter_kernel():
    mesh = plsc.VectorSubcoreMesh(core_axis_name="c", subcore_axis_name="s",
                                  num_cores=NC, num_subcores=NS)

    @functools.partial(
        pl.kernel,
        out_type=jax.ShapeDtypeStruct((NC, NP, F), jnp.float32),
        mesh=mesh,
        scratch_types=[
            pltpu.VMEM((CH // 2, K), jnp.int32),
            pltpu.VMEM((CH // 2, K), jnp.int32),
            pltpu.VMEM((K, F), jnp.float32),
            pltpu.VMEM((K, F), jnp.float32),
            pltpu.VMEM_SHARED((NP, F), jnp.float32),
            pltpu.SemaphoreType.DMA,
        ],
    )
    def scatter_kernel(src_idx, dst_idx, y_hbm, zeros_hbm, s_out,
                       sidx_v, didx_v, buf0_v, buf1_v, s_sh, gsem):
        c = lax.axis_index("c")
        s = lax.axis_index("s")
        hch = CH // 2

        pltpu.sync_copy(zeros_hbm, buf0_v)
        for k in range(RPT // K):
            pltpu.sync_copy(buf0_v, s_sh.at[pl.ds(s * RPT + k * K, K)])
        plsc.subcore_barrier()

        for h in range(2):
            pltpu.sync_copy(src_idx.at[c, s, pl.ds(h * hch, hch)], sidx_v)
            pltpu.sync_copy(dst_idx.at[c, s, pl.ds(h * hch, hch)], didx_v)
            pltpu.async_copy(y_hbm.at[sidx_v.at[0]], buf0_v, gsem)

            def body(j, carry):
                a = 2 * j
                b = a + 1
                pltpu.make_async_copy(y_hbm.at[sidx_v.at[a]], buf0_v,
                                      gsem).wait()
                pltpu.async_copy(y_hbm.at[sidx_v.at[b]], buf1_v, gsem)
                pltpu.sync_copy(buf0_v, s_sh.at[didx_v.at[a]], add=True)
                pltpu.make_async_copy(y_hbm.at[sidx_v.at[b]], buf1_v,
                                      gsem).wait()

                @pl.when(j + 1 < hch // 2)
                def _():
                    pltpu.async_copy(y_hbm.at[sidx_v.at[a + 2]], buf0_v, gsem)

                pltpu.sync_copy(buf1_v, s_sh.at[didx_v.at[b]], add=True)
                return carry

            lax.fori_loop(0, hch // 2, body, 0)
        plsc.subcore_barrier()
        for k in range(RPT // K):
            pltpu.sync_copy(s_sh.at[pl.ds(s * RPT + k * K, K)], buf0_v)
            pltpu.sync_copy(buf0_v, s_out.at[c, pl.ds(s * RPT + k * K, K)])

    return scatter_kernel


def _sc_scatter(src_idx, dst_idx, y, zeros):
    return _sc_scatter_kernel()(src_idx, dst_idx, y, zeros)


_BR = 256
_G = NP // _BR


def _dinv_block(deg_ref):
    d = deg_ref[0, :, 0:1] + deg_ref[1, :, 0:1] + 1.0
    return lax.rsqrt(d)


def _tc1_body(deg_ref, x_ref, w_ref, y_ref):
    dinv = _dinv_block(deg_ref)
    y_ref[...] = jnp.dot(x_ref[...], w_ref[...], precision=_HIGH) * dinv


def _tc2_body(deg_ref, sp_ref, y_ref, b_ref, w_ref, y2_ref):
    dinv = _dinv_block(deg_ref)
    st = sp_ref[0] + sp_ref[1] + y_ref[...]
    h = jnp.maximum(st * dinv + b_ref[...], 0.0)
    y2_ref[...] = jnp.dot(h, w_ref[...], precision=_HIGH) * dinv


def _tc3_body(deg_ref, sp_ref, y_ref, b_ref, w_ref, b3_ref, logp_ref, h_ref):
    dinv = _dinv_block(deg_ref)
    st = sp_ref[0] + sp_ref[1] + y_ref[...]
    h = jnp.maximum(st * dinv + b_ref[...], 0.0)
    h_ref[...] = h
    z = jnp.dot(h, w_ref[...], precision=_HIGH) + b3_ref[...]
    m = jnp.max(z, axis=1, keepdims=True)
    e = jnp.exp(z - m)
    lse = jnp.log(jnp.sum(e, axis=1, keepdims=True))
    logp_ref[...] = z - m - lse


def _row_spec(w):
    return pl.BlockSpec((_BR, w), lambda i: (i, 0))


_DEG_SPEC = pl.BlockSpec((NC, _BR, F), lambda i: (0, i, 0))
_SP_SPEC = pl.BlockSpec((NC, _BR, F), lambda i: (0, i, 0))


def _full(shape):
    return pl.BlockSpec(shape, lambda i: tuple(0 for _ in shape))


def _tc1(deg, x, w):
    return pl.pallas_call(
        _tc1_body,
        grid=(_G,),
        in_specs=[_DEG_SPEC, _row_spec(F), _full((F, F))],
        out_specs=_row_spec(F),
        out_shape=jax.ShapeDtypeStruct((NP, F), jnp.float32),
    )(deg, x, w)


def _tc2(deg, sp, y, b, w):
    return pl.pallas_call(
        _tc2_body,
        grid=(_G,),
        in_specs=[_DEG_SPEC, _SP_SPEC, _row_spec(F), _full((1, F)),
                  _full((F, F))],
        out_specs=_row_spec(F),
        out_shape=jax.ShapeDtypeStruct((NP, F), jnp.float32),
    )(deg, sp, y, b, w)


def _tc3(deg, sp, y, b, w, b3):
    return pl.pallas_call(
        _tc3_body,
        grid=(_G,),
        in_specs=[_DEG_SPEC, _SP_SPEC, _row_spec(F), _full((1, F)),
                  _full((F, C)), _full((1, C))],
        out_specs=[_row_spec(C), _row_spec(F)],
        out_shape=[jax.ShapeDtypeStruct((NP, C), jnp.float32),
                   jax.ShapeDtypeStruct((NP, F), jnp.float32)],
    )(deg, sp, y, b, w, b3)


def kernel(x, edge_index, W1, b1, W2, b2, W3, b3):
    src = jnp.pad(edge_index[0], (0, EP - E), constant_values=N)
    dst = jnp.pad(edge_index[1], (0, EP - E), constant_values=N)
    src_idx = src.reshape(NC, NS, CH, K)
    dst_idx = dst.reshape(NC, NS, CH, K)

    x_pad = jnp.pad(x, ((0, NP - N), (0, 0)))
    zerosF = jnp.zeros((K, F), jnp.float32)

    ones_tab = jnp.ones((NP, F), jnp.float32)
    deg = _sc_scatter(dst_idx, dst_idx, ones_tab, zerosF)
    y1 = _tc1(deg, x_pad, W1)
    s1 = _sc_scatter(src_idx, dst_idx, y1, zerosF)
    y2 = _tc2(deg, s1, y1, b1.reshape(1, F), W2)
    s2 = _sc_scatter(src_idx, dst_idx, y2, zerosF)
    logp, h = _tc3(deg, s2, y2, b2.reshape(1, F), W3, b3.reshape(1, C))
    return (logp[:N], h[:N])


# speedup vs baseline: 1.2815x; 1.2815x over previous
import functools

import jax
import jax.numpy as jnp
from jax import lax
from jax.experimental import pallas as pl
from jax.experimental.pallas import tpu as pltpu
from jax.experimental.pallas import tpu_sc as plsc

N = 10000
E = 320000
F = 128
C = 64

NC = 2
NS = 16
K = 128
CH = 80
EP = NC * NS * CH * K
NP = 10240
RPT = NP // NS
NB = 2

_HIGH = lax.Precision.HIGHEST


@functools.lru_cache(maxsize=None)
def _sc_degree_kernel():
    mesh = plsc.VectorSubcoreMesh(core_axis_name="c", subcore_axis_name="s",
                                  num_cores=NC, num_subcores=NS)

    @functools.partial(
        pl.kernel,
        out_type=jax.ShapeDtypeStruct((NC, NP, F), jnp.float32),
        mesh=mesh,
        scratch_types=[
            pltpu.VMEM((CH, K), jnp.int32),
            pltpu.VMEM((K, F), jnp.float32),
            pltpu.VMEM((K, F), jnp.float32),
            pltpu.VMEM_SHARED((NP, F), jnp.float32),
        ],
    )
    def deg_kernel(dst_idx, ones_hbm, zeros_hbm, deg_out,
                   didx_v, ones_v, g_v, deg_sh):
        c = lax.axis_index("c")
        s = lax.axis_index("s")
        pltpu.sync_copy(dst_idx.at[c, s], didx_v)
        pltpu.sync_copy(ones_hbm, ones_v)
        pltpu.sync_copy(zeros_hbm, g_v)
        for k in range(RPT // K):
            pltpu.sync_copy(g_v, deg_sh.at[pl.ds(s * RPT + k * K, K)])
        plsc.subcore_barrier()

        def body(j, carry):
            pltpu.sync_copy(ones_v, deg_sh.at[didx_v.at[j]], add=True)
            return carry

        lax.fori_loop(0, CH, body, 0)
        plsc.subcore_barrier()
        for k in range(RPT // K):
            pltpu.sync_copy(deg_sh.at[pl.ds(s * RPT + k * K, K)], g_v)
            pltpu.sync_copy(g_v, deg_out.at[c, pl.ds(s * RPT + k * K, K)])

    return deg_kernel


def _sc_degree(dst_idx, ones_hbm, zeros_hbm):
    return _sc_degree_kernel()(dst_idx, ones_hbm, zeros_hbm)


@functools.lru_cache(maxsize=None)
def _sc_scatter_kernel():
    mesh = plsc.VectorSubcoreMesh(core_axis_name="c", subcore_axis_name="s",
                                  num_cores=NC, num_subcores=NS)

    @functools.partial(
        pl.kernel,
        out_type=jax.ShapeDtypeStruct((NC, NP, F), jnp.float32),
        mesh=mesh,
        scratch_types=[
            pltpu.VMEM((CH // 2, K), jnp.int32),
            pltpu.VMEM((CH // 2, K), jnp.int32),
            pltpu.VMEM((K, F), jnp.float32),
            pltpu.VMEM((K, F), jnp.float32),
            pltpu.VMEM_SHARED((NP, F), jnp.float32),
            pltpu.SemaphoreType.DMA,
        ],
    )
    def scatter_kernel(src_idx, dst_idx, y_hbm, zeros_hbm, s_out,
                       sidx_v, didx_v, buf0_v, buf1_v, s_sh, gsem):
        c = lax.axis_index("c")
        s = lax.axis_index("s")
        hch = CH // 2

        pltpu.sync_copy(zeros_hbm, buf0_v)
        for k in range(RPT // K):
            pltpu.sync_copy(buf0_v, s_sh.at[pl.ds(s * RPT + k * K, K)])
        plsc.subcore_barrier()

        for h in range(2):
            pltpu.sync_copy(src_idx.at[c, s, pl.ds(h * hch, hch)], sidx_v)
            pltpu.sync_copy(dst_idx.at[c, s, pl.ds(h * hch, hch)], didx_v)
            pltpu.async_copy(y_hbm.at[sidx_v.at[0]], buf0_v, gsem)
            pltpu.async_copy(y_hbm.at[sidx_v.at[1]], buf1_v, gsem)

            def body(j, carry):
                a = 2 * j
                b = a + 1
                pltpu.make_async_copy(y_hbm.at[sidx_v.at[a]], buf0_v,
                                      gsem).wait()
                pltpu.sync_copy(buf0_v, s_sh.at[didx_v.at[a]], add=True)

                @pl.when(a + 2 < hch)
                def _():
                    pltpu.async_copy(y_hbm.at[sidx_v.at[a + 2]], buf0_v, gsem)

                pltpu.make_async_copy(y_hbm.at[sidx_v.at[b]], buf1_v,
                                      gsem).wait()
                pltpu.sync_copy(buf1_v, s_sh.at[didx_v.at[b]], add=True)

                @pl.when(b + 2 < hch)
                def _():
                    pltpu.async_copy(y_hbm.at[sidx_v.at[b + 2]], buf1_v, gsem)

                return carry

            lax.fori_loop(0, hch // 2, body, 0)
        plsc.subcore_barrier()
        for k in range(RPT // K):
            pltpu.sync_copy(s_sh.at[pl.ds(s * RPT + k * K, K)], buf0_v)
            pltpu.sync_copy(buf0_v, s_out.at[c, pl.ds(s * RPT + k * K, K)])

    return scatter_kernel


def _sc_scatter(src_idx, dst_idx, y, zeros):
    return _sc_scatter_kernel()(src_idx, dst_idx, y, zeros)


_BR = 256
_G = NP // _BR


def _dinv_block(deg_ref):
    d = deg_ref[0, :, 0:1] + deg_ref[1, :, 0:1] + 1.0
    return lax.rsqrt(d)


def _tc1_body(deg_ref, x_ref, w_ref, y_ref):
    dinv = _dinv_block(deg_ref)
    y_ref[...] = jnp.dot(x_ref[...], w_ref[...], precision=_HIGH) * dinv


def _tc2_body(deg_ref, sp_ref, y_ref, b_ref, w_ref, y2_ref):
    dinv = _dinv_block(deg_ref)
    st = sp_ref[0] + sp_ref[1] + y_ref[...]
    h = jnp.maximum(st * dinv + b_ref[...], 0.0)
    y2_ref[...] = jnp.dot(h, w_ref[...], precision=_HIGH) * dinv


def _tc3_body(deg_ref, sp_ref, y_ref, b_ref, w_ref, b3_ref, logp_ref, h_ref):
    dinv = _dinv_block(deg_ref)
    st = sp_ref[0] + sp_ref[1] + y_ref[...]
    h = jnp.maximum(st * dinv + b_ref[...], 0.0)
    h_ref[...] = h
    z = jnp.dot(h, w_ref[...], precision=_HIGH) + b3_ref[...]
    m = jnp.max(z, axis=1, keepdims=True)
    e = jnp.exp(z - m)
    lse = jnp.log(jnp.sum(e, axis=1, keepdims=True))
    logp_ref[...] = z - m - lse


def _row_spec(w):
    return pl.BlockSpec((_BR, w), lambda i: (i, 0))


_DEG_SPEC = pl.BlockSpec((NC, _BR, F), lambda i: (0, i, 0))
_SP_SPEC = pl.BlockSpec((NC, _BR, F), lambda i: (0, i, 0))


def _full(shape):
    return pl.BlockSpec(shape, lambda i: tuple(0 for _ in shape))


def _tc1(deg, x, w):
    return pl.pallas_call(
        _tc1_body,
        grid=(_G,),
        in_specs=[_DEG_SPEC, _row_spec(F), _full((F, F))],
        out_specs=_row_spec(F),
        out_shape=jax.ShapeDtypeStruct((NP, F), jnp.float32),
    )(deg, x, w)


def _tc2(deg, sp, y, b, w):
    return pl.pallas_call(
        _tc2_body,
        grid=(_G,),
        in_specs=[_DEG_SPEC, _SP_SPEC, _row_spec(F), _full((1, F)),
                  _full((F, F))],
        out_specs=_row_spec(F),
        out_shape=jax.ShapeDtypeStruct((NP, F), jnp.float32),
    )(deg, sp, y, b, w)


def _tc3(deg, sp, y, b, w, b3):
    return pl.pallas_call(
        _tc3_body,
        grid=(_G,),
        in_specs=[_DEG_SPEC, _SP_SPEC, _row_spec(F), _full((1, F)),
                  _full((F, C)), _full((1, C))],
        out_specs=[_row_spec(C), _row_spec(F)],
        out_shape=[jax.ShapeDtypeStruct((NP, C), jnp.float32),
                   jax.ShapeDtypeStruct((NP, F), jnp.float32)],
    )(deg, sp, y, b, w, b3)


def kernel(x, edge_index, W1, b1, W2, b2, W3, b3):
    src = jnp.pad(edge_index[0], (0, EP - E), constant_values=N)
    dst = jnp.pad(edge_index[1], (0, EP - E), constant_values=N)
    src_idx = src.reshape(NC, NS, CH, K)
    dst_idx = dst.reshape(NC, NS, CH, K)

    x_pad = jnp.pad(x, ((0, NP - N), (0, 0)))
    zerosF = jnp.zeros((K, F), jnp.float32)

    ones128 = jnp.ones((K, F), jnp.float32)
    deg = _sc_degree(dst_idx, ones128, zerosF)
    y1 = _tc1(deg, x_pad, W1)
    s1 = _sc_scatter(src_idx, dst_idx, y1, zerosF)
    y2 = _tc2(deg, s1, y1, b1.reshape(1, F), W2)
    s2 = _sc_scatter(src_idx, dst_idx, y2, zerosF)
    logp, h = _tc3(deg, s2, y2, b2.reshape(1, F), W3, b3.reshape(1, C))
    return (logp[:N], h[:N])
